# baseline (speedup 1.0000x reference)
import functools

import jax
import jax.numpy as jnp
from jax import lax
from jax.experimental import pallas as pl
from jax.experimental.pallas import tpu as pltpu
from jax.experimental.pallas import tpu_sc as plsc

_NC = 2
_NS = 16
_NW = _NC * _NS
_D = 128
_EPS = 1e-5
_CHUNK = 128
_N_ACC = 10112
_BE = 4096


def _gelu(t):
    return 0.5 * t * (1.0 + lax.erf(t * 0.7071067811865476))


def _ln_rows(t, g, b):
    m = jnp.mean(t, axis=-1, keepdims=True)
    v = jnp.mean((t - m) ** 2, axis=-1, keepdims=True)
    return g * (t - m) * lax.rsqrt(v + _EPS) + b


def _mm(a, w):
    return lax.dot_general(a, w, (((1,), (1,)), ((), ())),
                           preferred_element_type=jnp.float32)


def _node_mlp_kernel(x_ref, w1_ref, g1_ref, be1_ref, w2_ref, g2_ref, be2_ref,
                     o_ref):
    h = _mm(x_ref[...], w1_ref[...])
    m = jnp.mean(h, axis=0)
    v = jnp.mean((h - m) ** 2, axis=0)
    h = _gelu(g1_ref[...] * (h - m) * lax.rsqrt(v + _EPS) + be1_ref[...])
    h2 = _mm(h, w2_ref[...])
    m2 = jnp.mean(h2, axis=0)
    v2 = jnp.mean((h2 - m2) ** 2, axis=0)
    o_ref[...] = _gelu(g2_ref[...] * (h2 - m2) * lax.rsqrt(v2 + _EPS)
                       + be2_ref[...])


def _node_mlp(x, p):
    n = x.shape[0]
    return pl.pallas_call(
        _node_mlp_kernel,
        out_shape=jax.ShapeDtypeStruct((n, _D), jnp.float32),
    )(x, p['emb_W1'], p['emb_g1'].reshape(1, -1), p['emb_be1'].reshape(1, -1),
      p['emb_W2'], p['emb_g2'].reshape(1, -1), p['emb_be2'].reshape(1, -1))


def _proj_kernel(h_ref, wi_ref, wj_ref, oi_ref, oj_ref):
    oi_ref[...] = _mm(h_ref[...], wi_ref[...])
    oj_ref[...] = _mm(h_ref[...], wj_ref[...])


def _proj(h, wi, wj):
    n = h.shape[0]
    sh = jax.ShapeDtypeStruct((n, _D), jnp.float32)
    return pl.pallas_call(_proj_kernel, out_shape=(sh, sh))(h, wi, wj)


def _mm2_kernel(a_ref, b_ref, w_ref, o_ref):
    o_ref[...] = _mm(a_ref[...] + b_ref[...], w_ref[...])


def _mm2(a, b, w):
    n = a.shape[0]
    return pl.pallas_call(
        _mm2_kernel, out_shape=jax.ShapeDtypeStruct((n, _D), jnp.float32),
    )(a, b, w)


def _update_kernel(n, h_ref, ag_ref, wu1_ref, wu2_ref, bu_ref, gu_ref,
                   beu_ref, o_ref):
    h = h_ref[...]
    ag = ag_ref[...]
    agg = ag[0, :n] + ag[1, :n]
    t = _mm(h, wu1_ref[...]) + _mm(agg, wu2_ref[...]) + bu_ref[...]
    o_ref[...] = h + _gelu(_ln_rows(t, gu_ref[...], beu_ref[...]))


def _update(h, agg2, lp):
    n = h.shape[0]
    return pl.pallas_call(
        functools.partial(_update_kernel, n),
        out_shape=jax.ShapeDtypeStruct((n, _D), jnp.float32),
    )(h, agg2, lp['Wu'][:, :_D], lp['Wu'][:, _D:],
      lp['bu'].reshape(1, -1), lp['gu'].reshape(1, -1),
      lp['beu'].reshape(1, -1))


def _msg_kernel(a_ref, b_ref, bm_ref, gm_ref, bem_ref, o_ref):
    t = a_ref[...] + b_ref[...] + bm_ref[...]
    o_ref[...] = _gelu(_ln_rows(t, gm_ref[...], bem_ref[...]))


def _msg(ga, gb, lp):
    e = ga.shape[0]
    grid = e // _BE
    blk = pl.BlockSpec((_BE, _D), lambda i: (i, 0))
    par = pl.BlockSpec((1, _D), lambda i: (0, 0))
    return pl.pallas_call(
        _msg_kernel,
        grid=(grid,),
        in_specs=[blk, blk, par, par, par],
        out_specs=blk,
        out_shape=jax.ShapeDtypeStruct((e, _D), jnp.float32),
    )(ga, gb, lp['bm'].reshape(1, -1), lp['gm'].reshape(1, -1),
      lp['bem'].reshape(1, -1))


def _stats_kernel(a_ref, b_ref, o_ref):
    q = a_ref[...] - b_ref[...]
    blk = jnp.concatenate(
        [jnp.sum(q, axis=0, keepdims=True),
         jnp.sum(q * q, axis=0, keepdims=True)], axis=0)

    @pl.when(pl.program_id(0) == 0)
    def _init():
        o_ref[...] = jnp.zeros_like(o_ref)

    o_ref[...] += blk


def _stats(ga, gb):
    e = ga.shape[0]
    blk = pl.BlockSpec((_BE, _D), lambda i: (i, 0))
    return pl.pallas_call(
        _stats_kernel,
        grid=(e // _BE,),
        in_specs=[blk, blk],
        out_specs=pl.BlockSpec((2, _D), lambda i: (0, 0)),
        out_shape=jax.ShapeDtypeStruct((2, _D), jnp.float32),
    )(ga, gb)


def _final_kernel(n_real, a_ref, b_ref, st_ref, g_ref, be_ref, w2_ref, b2_ref,
                  o_ref):
    q = a_ref[...] - b_ref[...]
    st = st_ref[...]
    m = st[0:1] * (1.0 / n_real)
    v = st[1:2] * (1.0 / n_real) - m * m
    t = _gelu(g_ref[...] * (q - m) * lax.rsqrt(v + _EPS) + be_ref[...])
    o_ref[...] = _mm(t, w2_ref[...]) + b2_ref[...]


def _final(ga, gb, st, p, n_real):
    e = ga.shape[0]
    blk = pl.BlockSpec((_BE, _D), lambda i: (i, 0))
    par = pl.BlockSpec((1, _D), lambda i: (0, 0))
    w2p = jnp.zeros((8, _D), jnp.float32).at[:3].set(p['fin_W2'])
    b2p = jnp.zeros((1, 8), jnp.float32).at[0, :3].set(p['fin_b2'])
    return pl.pallas_call(
        functools.partial(_final_kernel, float(n_real)),
        grid=(e // _BE,),
        in_specs=[blk, blk,
                  pl.BlockSpec((2, _D), lambda i: (0, 0)), par, par,
                  pl.BlockSpec((8, _D), lambda i: (0, 0)),
                  pl.BlockSpec((1, 8), lambda i: (0, 0))],
        out_specs=pl.BlockSpec((_BE, 8), lambda i: (i, 0)),
        out_shape=jax.ShapeDtypeStruct((e, 8), jnp.float32),
    )(ga, gb, st, p['fin_g1'].reshape(1, -1), p['fin_be1'].reshape(1, -1),
      w2p, b2p)


def _sc_mesh():
    return plsc.VectorSubcoreMesh(core_axis_name="c", subcore_axis_name="s",
                                  num_cores=_NC, num_subcores=_NS)


_Q0 = 104
_Q1 = 56


def _sc_gather2(ta, tb, ia, ib):
    e = ia.shape[0]
    assert e == _NS * (_Q0 + _Q1) * _CHUNK
    qmax = max(_Q0, _Q1)
    sh = jax.ShapeDtypeStruct((e, _D), jnp.float32)

    @functools.partial(
        pl.kernel,
        out_type=(sh, sh),
        mesh=_sc_mesh(),
        scratch_types=[
            pltpu.VMEM((qmax * _CHUNK,), jnp.int32),
            pltpu.VMEM((qmax * _CHUNK,), jnp.int32),
            pltpu.VMEM((_CHUNK, _D), jnp.float32),
            pltpu.VMEM((_CHUNK, _D), jnp.float32),
            pltpu.VMEM((_CHUNK, _D), jnp.float32),
            pltpu.VMEM((_CHUNK, _D), jnp.float32),
            pltpu.SemaphoreType.DMA,
            pltpu.SemaphoreType.DMA,
            pltpu.SemaphoreType.DMA,
            pltpu.SemaphoreType.DMA,
        ],
    )
    def k(ta_h, tb_h, ia_h, ib_h, oa_h, ob_h,
          iav, ibv, ra0, ra1, rb0, rb1, sg0, sg1, sw0, sw1):
        ra = (ra0, ra1)
        rb = (rb0, rb1)
        sg = (sg0, sg1)
        sw = (sw0, sw1)
        c = lax.axis_index("c")
        s = lax.axis_index("s")

        def drain2(sem):
            pltpu.make_async_copy(ta_h.at[pl.ds(0, _CHUNK)], ra0, sem).wait()
            pltpu.make_async_copy(ta_h.at[pl.ds(0, _CHUNK)], rb0, sem).wait()

        def run(base_e, n_my):
            pltpu.sync_copy(ia_h.at[pl.ds(base_e, n_my * _CHUNK)],
                            iav.at[pl.ds(0, n_my * _CHUNK)])
            pltpu.sync_copy(ib_h.at[pl.ds(base_e, n_my * _CHUNK)],
                            ibv.at[pl.ds(0, n_my * _CHUNK)])

            def start_g(i, b):
                pltpu.async_copy(ta_h.at[iav.at[pl.ds(i * _CHUNK, _CHUNK)]],
                                 ra[b], sg[b])
                pltpu.async_copy(tb_h.at[ibv.at[pl.ds(i * _CHUNK, _CHUNK)]],
                                 rb[b], sg[b])

            def start_wb(i, b):
                base = base_e + i * _CHUNK
                pltpu.async_copy(ra[b], oa_h.at[pl.ds(base, _CHUNK)], sw[b])
                pltpu.async_copy(rb[b], ob_h.at[pl.ds(base, _CHUNK)], sw[b])

            n_pair = n_my // 2
            start_g(0, 0)

            def pair(p, carry):
                @pl.when(p > 0)
                def _():
                    drain2(sw[1])
                start_g(2 * p + 1, 1)
                drain2(sg[0])
                start_wb(2 * p, 0)
                drain2(sw[0])
                @pl.when(p + 1 < n_pair)
                def _():
                    start_g(2 * p + 2, 0)
                drain2(sg[1])
                start_wb(2 * p + 1, 1)
                return carry

            lax.fori_loop(0, n_pair, pair, 0)
            drain2(sw[1])

        @pl.when(c == 0)
        def _():
            run(s * _Q0 * _CHUNK, _Q0)

        @pl.when(c == 1)
        def _():
            run((_NS * _Q0 + s * _Q1) * _CHUNK, _Q1)

    return k(ta, tb, ia, ib)


def _sc_scatter_add(msg, dsts, zrows):
    e = msg.shape[0]
    per_w = e // _NW
    n_ch = per_w // _CHUNK
    zc = _N_ACC // _NS

    @functools.partial(
        pl.kernel,
        out_type=jax.ShapeDtypeStruct((_NC, _N_ACC, _D), jnp.float32),
        mesh=_sc_mesh(),
        scratch_types=[
            pltpu.VMEM((_CHUNK,), jnp.int32),
            pltpu.VMEM((_CHUNK,), jnp.int32),
            pltpu.VMEM((_CHUNK, _D), jnp.float32),
            pltpu.VMEM((_CHUNK, _D), jnp.float32),
            pltpu.SemaphoreType.DMA,
            pltpu.SemaphoreType.DMA,
            pltpu.SemaphoreType.DMA,
            pltpu.SemaphoreType.DMA,
            pltpu.VMEM_SHARED((_N_ACC, _D), jnp.float32),
        ],
    )
    def k(msg_h, dst_h, z_h, out_h, idx0, idx1, rows0, rows1,
          sl0, sl1, ss0, ss1, shared):
        idx = (idx0, idx1)
        rows = (rows0, rows1)
        sl = (sl0, sl1)
        ss = (ss0, ss1)
        c = lax.axis_index("c")
        s = lax.axis_index("s")
        wid = s * _NC + c
        pltpu.sync_copy(z_h.at[pl.ds(s * zc, zc)], shared.at[pl.ds(s * zc, zc)])
        plsc.subcore_barrier()
        base_w = wid * per_w
        n_pair = n_ch // 2

        def start_load(i, b):
            base = base_w + i * _CHUNK
            pltpu.async_copy(dst_h.at[pl.ds(base, _CHUNK)], idx[b], sl[b])
            pltpu.async_copy(msg_h.at[pl.ds(base, _CHUNK)], rows[b], sl[b])

        def drain_load(b):
            pltpu.make_async_copy(dst_h.at[pl.ds(0, _CHUNK)], idx0, sl[b]).wait()
            pltpu.make_async_copy(msg_h.at[pl.ds(0, _CHUNK)], rows0, sl[b]).wait()

        def start_scat(b):
            pltpu.async_copy(rows[b], shared.at[idx[b]], ss[b], add=True)

        def drain_scat(b):
            pltpu.make_async_copy(msg_h.at[pl.ds(0, _CHUNK)], rows0, ss[b]).wait()

        start_load(0, 0)

        def pair(p, carry):
            @pl.when(p > 0)
            def _():
                drain_scat(1)
            start_load(2 * p + 1, 1)
            drain_load(0)
            start_scat(0)
            drain_scat(0)
            @pl.when(p + 1 < n_pair)
            def _():
                start_load(2 * p + 2, 0)
            drain_load(1)
            start_scat(1)
            return carry

        lax.fori_loop(0, n_pair, pair, 0)
        drain_scat(1)
        plsc.subcore_barrier()
        pltpu.sync_copy(shared.at[pl.ds(s * zc, zc)],
                        out_h.at[c, pl.ds(s * zc, zc)])

    return k(msg, dsts, zrows)


def kernel(x, params, edge_index):
    p = params
    n = x.shape[0]
    e = edge_index.shape[1]
    n_ch = (e + _NW * _CHUNK - 1) // (_NW * _CHUNK)
    n_ch += n_ch % 2
    e_pad = _NW * _CHUNK * n_ch
    src = edge_index[0].astype(jnp.int32)
    dst = edge_index[1].astype(jnp.int32)
    pad0 = jnp.zeros((e_pad - e,), jnp.int32)
    ia = jnp.concatenate([dst, pad0])
    ib = jnp.concatenate([src, pad0])
    dsts = jnp.concatenate([dst, jnp.full((e_pad - e,), n, jnp.int32)])
    zrows = jnp.zeros((_N_ACC, _D), jnp.float32)

    h = _node_mlp(x, p)
    h0 = h
    for lp in p['gnn']:
        ai, aj = _proj(h, lp['Wm'][:, :_D], lp['Wm'][:, _D:])
        ga, gb = _sc_gather2(ai, aj, ia, ib)
        msg = _msg(ga, gb, lp)
        agg2 = _sc_scatter_add(msg, dsts, zrows)
        h = _update(h, agg2, lp)

    pfin = _mm2(h, h0, p['fin_W1'])
    ga, gb = _sc_gather2(pfin, pfin, ia, ib)
    st = _stats(ga, gb)
    out8 = _final(ga, gb, st, p, e)
    return out8[:e, :3]


# speedup vs baseline: 1.0790x; 1.0790x over previous
import functools

import jax
import jax.numpy as jnp
from jax import lax
from jax.experimental import pallas as pl
from jax.experimental.pallas import tpu as pltpu
from jax.experimental.pallas import tpu_sc as plsc

_NC = 2
_NS = 16
_NW = _NC * _NS
_D = 128
_EPS = 1e-5
_CHUNK = 128
_N_ACC = 10112
_BE = 4096


def _pack(r):
    lo = lax.bitcast_convert_type(r[:, :64].astype(jnp.bfloat16), jnp.uint16)
    hi = lax.bitcast_convert_type(r[:, 64:].astype(jnp.bfloat16), jnp.uint16)
    return (hi.astype(jnp.uint32) << 16) | lo.astype(jnp.uint32)


def _unpack(a32):
    lo = lax.bitcast_convert_type(a32 << 16, jnp.float32)
    hi = lax.bitcast_convert_type(a32 & jnp.uint32(0xFFFF0000), jnp.float32)
    return jnp.concatenate([lo, hi], axis=-1)


def _gelu(t):
    return 0.5 * t * (1.0 + lax.erf(t * 0.7071067811865476))


def _ln_rows(t, g, b):
    m = jnp.mean(t, axis=-1, keepdims=True)
    v = jnp.mean((t - m) ** 2, axis=-1, keepdims=True)
    return g * (t - m) * lax.rsqrt(v + _EPS) + b


def _mm(a, w):
    return lax.dot_general(a, w, (((1,), (1,)), ((), ())),
                           preferred_element_type=jnp.float32)


def _node_mlp_kernel(x_ref, w1_ref, g1_ref, be1_ref, w2_ref, g2_ref, be2_ref,
                     o_ref):
    h = _mm(x_ref[...], w1_ref[...])
    m = jnp.mean(h, axis=0)
    v = jnp.mean((h - m) ** 2, axis=0)
    h = _gelu(g1_ref[...] * (h - m) * lax.rsqrt(v + _EPS) + be1_ref[...])
    h2 = _mm(h, w2_ref[...])
    m2 = jnp.mean(h2, axis=0)
    v2 = jnp.mean((h2 - m2) ** 2, axis=0)
    o_ref[...] = _gelu(g2_ref[...] * (h2 - m2) * lax.rsqrt(v2 + _EPS)
                       + be2_ref[...])


def _node_mlp(x, p):
    n = x.shape[0]
    return pl.pallas_call(
        _node_mlp_kernel,
        out_shape=jax.ShapeDtypeStruct((n, _D), jnp.float32),
    )(x, p['emb_W1'], p['emb_g1'].reshape(1, -1), p['emb_be1'].reshape(1, -1),
      p['emb_W2'], p['emb_g2'].reshape(1, -1), p['emb_be2'].reshape(1, -1))


def _proj_kernel(h_ref, wi_ref, wj_ref, oi_ref, oj_ref):
    oi_ref[...] = _pack(_mm(h_ref[...], wi_ref[...]))
    oj_ref[...] = _pack(_mm(h_ref[...], wj_ref[...]))


def _proj(h, wi, wj):
    n = h.shape[0]
    sh = jax.ShapeDtypeStruct((n, _D // 2), jnp.uint32)
    return pl.pallas_call(_proj_kernel, out_shape=(sh, sh))(h, wi, wj)


def _mm2_kernel(a_ref, b_ref, w_ref, o_ref):
    o_ref[...] = _pack(_mm(a_ref[...] + b_ref[...], w_ref[...]))


def _mm2(a, b, w):
    n = a.shape[0]
    return pl.pallas_call(
        _mm2_kernel, out_shape=jax.ShapeDtypeStruct((n, _D // 2), jnp.uint32),
    )(a, b, w)


def _update_kernel(n, h_ref, ag_ref, wu1_ref, wu2_ref, bu_ref, gu_ref,
                   beu_ref, o_ref):
    h = h_ref[...]
    ag = ag_ref[...]
    agg = ag[0, :n] + ag[1, :n]
    t = _mm(h, wu1_ref[...]) + _mm(agg, wu2_ref[...]) + bu_ref[...]
    o_ref[...] = h + _gelu(_ln_rows(t, gu_ref[...], beu_ref[...]))


def _update(h, agg2, lp, perm):
    n = h.shape[0]
    return pl.pallas_call(
        functools.partial(_update_kernel, n),
        out_shape=jax.ShapeDtypeStruct((n, _D), jnp.float32),
    )(h, agg2, lp['Wu'][:, :_D], lp['Wu'][:, _D:][:, perm],
      lp['bu'].reshape(1, -1), lp['gu'].reshape(1, -1),
      lp['beu'].reshape(1, -1))


def _msg_kernel(a_ref, b_ref, bm_ref, gm_ref, bem_ref, o_ref):
    t = _unpack(a_ref[...]) + _unpack(b_ref[...]) + bm_ref[...]
    o_ref[...] = _gelu(_ln_rows(t, gm_ref[...], bem_ref[...]))


def _msg(ga, gb, bm_p, gm_p, bem_p):
    e = ga.shape[0]
    grid = e // _BE
    blkp = pl.BlockSpec((_BE, _D // 2), lambda i: (i, 0))
    blk = pl.BlockSpec((_BE, _D), lambda i: (i, 0))
    par = pl.BlockSpec((1, _D), lambda i: (0, 0))
    return pl.pallas_call(
        _msg_kernel,
        grid=(grid,),
        in_specs=[blkp, blkp, par, par, par],
        out_specs=blk,
        out_shape=jax.ShapeDtypeStruct((e, _D), jnp.float32),
    )(ga, gb, bm_p.reshape(1, -1), gm_p.reshape(1, -1),
      bem_p.reshape(1, -1))


def _stats_kernel(a_ref, b_ref, o_ref):
    q = _unpack(a_ref[...]) - _unpack(b_ref[...])
    blk = jnp.concatenate(
        [jnp.sum(q, axis=0, keepdims=True),
         jnp.sum(q * q, axis=0, keepdims=True)], axis=0)

    @pl.when(pl.program_id(0) == 0)
    def _init():
        o_ref[...] = jnp.zeros_like(o_ref)

    o_ref[...] += blk


def _stats(ga, gb):
    e = ga.shape[0]
    blkp = pl.BlockSpec((_BE, _D // 2), lambda i: (i, 0))
    return pl.pallas_call(
        _stats_kernel,
        grid=(e // _BE,),
        in_specs=[blkp, blkp],
        out_specs=pl.BlockSpec((2, _D), lambda i: (0, 0)),
        out_shape=jax.ShapeDtypeStruct((2, _D), jnp.float32),
    )(ga, gb)


def _final_kernel(n_real, a_ref, b_ref, st_ref, g_ref, be_ref, w2_ref, b2_ref,
                  o_ref):
    q = _unpack(a_ref[...]) - _unpack(b_ref[...])
    st = st_ref[...]
    m = st[0:1] * (1.0 / n_real)
    v = st[1:2] * (1.0 / n_real) - m * m
    t = _gelu(g_ref[...] * (q - m) * lax.rsqrt(v + _EPS) + be_ref[...])
    o_ref[...] = _mm(t, w2_ref[...]) + b2_ref[...]


def _final(ga, gb, st, p, perm, n_real):
    e = ga.shape[0]
    blkp = pl.BlockSpec((_BE, _D // 2), lambda i: (i, 0))
    par = pl.BlockSpec((1, _D), lambda i: (0, 0))
    w2p = jnp.zeros((8, _D), jnp.float32).at[:3].set(p['fin_W2'][:, perm])
    b2p = jnp.zeros((1, 8), jnp.float32).at[0, :3].set(p['fin_b2'])
    return pl.pallas_call(
        functools.partial(_final_kernel, float(n_real)),
        grid=(e // _BE,),
        in_specs=[blkp, blkp,
                  pl.BlockSpec((2, _D), lambda i: (0, 0)), par, par,
                  pl.BlockSpec((8, _D), lambda i: (0, 0)),
                  pl.BlockSpec((1, 8), lambda i: (0, 0))],
        out_specs=pl.BlockSpec((_BE, 8), lambda i: (i, 0)),
        out_shape=jax.ShapeDtypeStruct((e, 8), jnp.float32),
    )(ga, gb, st, p['fin_g1'][perm].reshape(1, -1),
      p['fin_be1'][perm].reshape(1, -1), w2p, b2p)


def _sc_mesh():
    return plsc.VectorSubcoreMesh(core_axis_name="c", subcore_axis_name="s",
                                  num_cores=_NC, num_subcores=_NS)


_Q0 = 80
_Q1 = 80


def _sc_gather2(ta, tb, ia, ib):
    e = ia.shape[0]
    assert e == _NS * (_Q0 + _Q1) * _CHUNK
    qmax = max(_Q0, _Q1)
    sh = jax.ShapeDtypeStruct((e, _D // 2), jnp.uint32)

    @functools.partial(
        pl.kernel,
        out_type=(sh, sh),
        mesh=_sc_mesh(),
        compiler_params=pltpu.CompilerParams(use_tc_tiling_on_sc=False),
        scratch_types=[
            pltpu.VMEM((qmax * _CHUNK,), jnp.int32),
            pltpu.VMEM((qmax * _CHUNK,), jnp.int32),
            pltpu.VMEM((_CHUNK, _D // 2), jnp.uint32),
            pltpu.VMEM((_CHUNK, _D // 2), jnp.uint32),
            pltpu.VMEM((_CHUNK, _D // 2), jnp.uint32),
            pltpu.VMEM((_CHUNK, _D // 2), jnp.uint32),
            pltpu.SemaphoreType.DMA,
            pltpu.SemaphoreType.DMA,
            pltpu.SemaphoreType.DMA,
            pltpu.SemaphoreType.DMA,
        ],
    )
    def k(ta_h, tb_h, ia_h, ib_h, oa_h, ob_h,
          iav, ibv, ra0, ra1, rb0, rb1, sg0, sg1, sw0, sw1):
        ra = (ra0, ra1)
        rb = (rb0, rb1)
        sg = (sg0, sg1)
        sw = (sw0, sw1)
        c = lax.axis_index("c")
        s = lax.axis_index("s")

        def drain2(sem):
            pltpu.make_async_copy(ta_h.at[pl.ds(0, _CHUNK)], ra0, sem).wait()
            pltpu.make_async_copy(ta_h.at[pl.ds(0, _CHUNK)], rb0, sem).wait()

        def run(base_e, n_my):
            pltpu.sync_copy(ia_h.at[pl.ds(base_e, n_my * _CHUNK)],
                            iav.at[pl.ds(0, n_my * _CHUNK)])
            pltpu.sync_copy(ib_h.at[pl.ds(base_e, n_my * _CHUNK)],
                            ibv.at[pl.ds(0, n_my * _CHUNK)])

            def start_g(i, b):
                pltpu.async_copy(ta_h.at[iav.at[pl.ds(i * _CHUNK, _CHUNK)]],
                                 ra[b], sg[b])
                pltpu.async_copy(tb_h.at[ibv.at[pl.ds(i * _CHUNK, _CHUNK)]],
                                 rb[b], sg[b])

            def start_wb(i, b):
                base = base_e + i * _CHUNK
                pltpu.async_copy(ra[b], oa_h.at[pl.ds(base, _CHUNK)], sw[b])
                pltpu.async_copy(rb[b], ob_h.at[pl.ds(base, _CHUNK)], sw[b])

            n_pair = n_my // 2
            start_g(0, 0)

            def pair(p, carry):
                @pl.when(p > 0)
                def _():
                    drain2(sw[1])
                start_g(2 * p + 1, 1)
                drain2(sg[0])
                start_wb(2 * p, 0)
                drain2(sw[0])
                @pl.when(p + 1 < n_pair)
                def _():
                    start_g(2 * p + 2, 0)
                drain2(sg[1])
                start_wb(2 * p + 1, 1)
                return carry

            lax.fori_loop(0, n_pair, pair, 0)
            drain2(sw[1])

        @pl.when(c == 0)
        def _():
            run(s * _Q0 * _CHUNK, _Q0)

        @pl.when(c == 1)
        def _():
            run((_NS * _Q0 + s * _Q1) * _CHUNK, _Q1)

    return k(ta, tb, ia, ib)


def _sc_scatter_add(msg, dsts, zrows):
    e = msg.shape[0]
    per_w = e // _NW
    n_ch = per_w // _CHUNK
    zc = _N_ACC // _NS

    @functools.partial(
        pl.kernel,
        out_type=jax.ShapeDtypeStruct((_NC, _N_ACC, _D), jnp.float32),
        mesh=_sc_mesh(),
        scratch_types=[
            pltpu.VMEM((_CHUNK,), jnp.int32),
            pltpu.VMEM((_CHUNK,), jnp.int32),
            pltpu.VMEM((_CHUNK, _D), jnp.float32),
            pltpu.VMEM((_CHUNK, _D), jnp.float32),
            pltpu.SemaphoreType.DMA,
            pltpu.SemaphoreType.DMA,
            pltpu.SemaphoreType.DMA,
            pltpu.SemaphoreType.DMA,
            pltpu.VMEM_SHARED((_N_ACC, _D), jnp.float32),
        ],
    )
    def k(msg_h, dst_h, z_h, out_h, idx0, idx1, rows0, rows1,
          sl0, sl1, ss0, ss1, shared):
        idx = (idx0, idx1)
        rows = (rows0, rows1)
        sl = (sl0, sl1)
        ss = (ss0, ss1)
        c = lax.axis_index("c")
        s = lax.axis_index("s")
        wid = s * _NC + c
        pltpu.sync_copy(z_h.at[pl.ds(s * zc, zc)], shared.at[pl.ds(s * zc, zc)])
        plsc.subcore_barrier()
        base_w = wid * per_w
        n_pair = n_ch // 2

        def start_load(i, b):
            base = base_w + i * _CHUNK
            pltpu.async_copy(dst_h.at[pl.ds(base, _CHUNK)], idx[b], sl[b])
            pltpu.async_copy(msg_h.at[pl.ds(base, _CHUNK)], rows[b], sl[b])

        def drain_load(b):
            pltpu.make_async_copy(dst_h.at[pl.ds(0, _CHUNK)], idx0, sl[b]).wait()
            pltpu.make_async_copy(msg_h.at[pl.ds(0, _CHUNK)], rows0, sl[b]).wait()

        def start_scat(b):
            pltpu.async_copy(rows[b], shared.at[idx[b]], ss[b], add=True)

        def drain_scat(b):
            pltpu.make_async_copy(msg_h.at[pl.ds(0, _CHUNK)], rows0, ss[b]).wait()

        start_load(0, 0)

        def pair(p, carry):
            @pl.when(p > 0)
            def _():
                drain_scat(1)
            start_load(2 * p + 1, 1)
            drain_load(0)
            start_scat(0)
            drain_scat(0)
            @pl.when(p + 1 < n_pair)
            def _():
                start_load(2 * p + 2, 0)
            drain_load(1)
            start_scat(1)
            return carry

        lax.fori_loop(0, n_pair, pair, 0)
        drain_scat(1)
        plsc.subcore_barrier()
        pltpu.sync_copy(shared.at[pl.ds(s * zc, zc)],
                        out_h.at[c, pl.ds(s * zc, zc)])

    return k(msg, dsts, zrows)


def kernel(x, params, edge_index):
    p = params
    n = x.shape[0]
    e = edge_index.shape[1]
    n_ch = (e + _NW * _CHUNK - 1) // (_NW * _CHUNK)
    n_ch += n_ch % 2
    e_pad = _NW * _CHUNK * n_ch
    src = edge_index[0].astype(jnp.int32)
    dst = edge_index[1].astype(jnp.int32)
    pad0 = jnp.zeros((e_pad - e,), jnp.int32)
    ia = jnp.concatenate([dst, pad0])
    ib = jnp.concatenate([src, pad0])
    dsts = jnp.concatenate([dst, jnp.full((e_pad - e,), n, jnp.int32)])
    zrows = jnp.zeros((_N_ACC, _D), jnp.float32)

    perm = jnp.concatenate([jnp.arange(0, _D, 2), jnp.arange(1, _D, 2)])
    h = _node_mlp(x, p)
    h0 = h
    for lp in p['gnn']:
        ai, aj = _proj(h, lp['Wm'][:, :_D][perm, :], lp['Wm'][:, _D:][perm, :])
        ga, gb = _sc_gather2(ai, aj, ia, ib)
        msg = _msg(ga, gb, lp['bm'][perm], lp['gm'][perm], lp['bem'][perm])
        agg2 = _sc_scatter_add(msg, dsts, zrows)
        h = _update(h, agg2, lp, perm)

    pfin = _mm2(h, h0, p['fin_W1'][perm, :])
    ga, gb = _sc_gather2(pfin, pfin, ia, ib)
    st = _stats(ga, gb)
    out8 = _final(ga, gb, st, p, perm, e)
    return out8[:e, :3]


# speedup vs baseline: 1.2604x; 1.1681x over previous
import functools

import jax
import jax.numpy as jnp
from jax import lax
from jax.experimental import pallas as pl
from jax.experimental.pallas import tpu as pltpu
from jax.experimental.pallas import tpu_sc as plsc

_NC = 2
_NS = 16
_NW = _NC * _NS
_D = 128
_EPS = 1e-5
_CHUNK = 128
_N_ACC = 10112
_BE = 4096
_BER = 2048


def _pack(r):
    lo = lax.bitcast_convert_type(r[:, :64].astype(jnp.bfloat16), jnp.uint16)
    hi = lax.bitcast_convert_type(r[:, 64:].astype(jnp.bfloat16), jnp.uint16)
    return (hi.astype(jnp.uint32) << 16) | lo.astype(jnp.uint32)


def _unpack2(a32):
    lo = lax.bitcast_convert_type(a32 << 16, jnp.float32)
    hi = lax.bitcast_convert_type(a32 & jnp.uint32(0xFFFF0000), jnp.float32)
    e0 = jnp.concatenate([lo[:, :64], hi[:, :64]], axis=-1)
    e1 = jnp.concatenate([lo[:, 64:], hi[:, 64:]], axis=-1)
    return e0, e1


def _ilv(x0, x1):
    m, d = x0.shape
    return jnp.stack([x0, x1], axis=1).reshape(2 * m, d)


def _gelu(t):
    return 0.5 * t * (1.0 + lax.erf(t * 0.7071067811865476))


def _ln_rows(t, g, b):
    m = jnp.mean(t, axis=-1, keepdims=True)
    v = jnp.mean((t - m) ** 2, axis=-1, keepdims=True)
    return g * (t - m) * lax.rsqrt(v + _EPS) + b


def _mm(a, w):
    return lax.dot_general(a, w, (((1,), (1,)), ((), ())),
                           preferred_element_type=jnp.float32)


def _node_mlp_kernel(x_ref, w1_ref, g1_ref, be1_ref, w2_ref, g2_ref, be2_ref,
                     o_ref):
    h = _mm(x_ref[...], w1_ref[...])
    m = jnp.mean(h, axis=0)
    v = jnp.mean((h - m) ** 2, axis=0)
    h = _gelu(g1_ref[...] * (h - m) * lax.rsqrt(v + _EPS) + be1_ref[...])
    h2 = _mm(h, w2_ref[...])
    m2 = jnp.mean(h2, axis=0)
    v2 = jnp.mean((h2 - m2) ** 2, axis=0)
    o_ref[...] = _gelu(g2_ref[...] * (h2 - m2) * lax.rsqrt(v2 + _EPS)
                       + be2_ref[...])


def _node_mlp(x, p):
    n = x.shape[0]
    return pl.pallas_call(
        _node_mlp_kernel,
        out_shape=jax.ShapeDtypeStruct((n, _D), jnp.float32),
    )(x, p['emb_W1'], p['emb_g1'].reshape(1, -1), p['emb_be1'].reshape(1, -1),
      p['emb_W2'], p['emb_g2'].reshape(1, -1), p['emb_be2'].reshape(1, -1))


def _proj_kernel(h_ref, wi_ref, wj_ref, oi_ref, oj_ref):
    oi_ref[...] = _pack(_mm(h_ref[...], wi_ref[...]))
    oj_ref[...] = _pack(_mm(h_ref[...], wj_ref[...]))


def _proj(h, wi, wj):
    n = h.shape[0]
    sh = jax.ShapeDtypeStruct((n, _D // 2), jnp.uint32)
    return pl.pallas_call(_proj_kernel, out_shape=(sh, sh))(h, wi, wj)


def _mm2_kernel(a_ref, b_ref, w_ref, o_ref):
    o_ref[...] = _pack(_mm(a_ref[...] + b_ref[...], w_ref[...]))


def _mm2(a, b, w):
    n = a.shape[0]
    return pl.pallas_call(
        _mm2_kernel, out_shape=jax.ShapeDtypeStruct((n, _D // 2), jnp.uint32),
    )(a, b, w)


def _update_kernel(n, h_ref, ag_ref, wu1_ref, wu2_ref, bu_ref, gu_ref,
                   beu_ref, o_ref):
    h = h_ref[...]
    ag = ag_ref[...]
    agg = ag[0, :n] + ag[1, :n]
    t = _mm(h, wu1_ref[...]) + _mm(agg, wu2_ref[...]) + bu_ref[...]
    o_ref[...] = h + _gelu(_ln_rows(t, gu_ref[...], beu_ref[...]))


def _update(h, agg2, lp, perm):
    n = h.shape[0]
    return pl.pallas_call(
        functools.partial(_update_kernel, n),
        out_shape=jax.ShapeDtypeStruct((n, _D), jnp.float32),
    )(h, agg2, lp['Wu'][:, :_D], lp['Wu'][:, _D:][:, perm],
      lp['bu'].reshape(1, -1), lp['gu'].reshape(1, -1),
      lp['beu'].reshape(1, -1))


def _msg_kernel(a_ref, b_ref, bm_ref, gm_ref, bem_ref, o_ref):
    a0, a1 = _unpack2(a_ref[...])
    b0, b1 = _unpack2(b_ref[...])
    gm = gm_ref[...]
    bem = bem_ref[...]
    m0 = _gelu(_ln_rows(a0 + b0 + bm_ref[...], gm, bem))
    m1 = _gelu(_ln_rows(a1 + b1 + bm_ref[...], gm, bem))
    o_ref[...] = _ilv(m0, m1)


def _msg(ga2, gb2, bm_p, gm_p, bem_p):
    eh = ga2.shape[0]
    grid = eh // _BER
    blkp = pl.BlockSpec((_BER, _D), lambda i: (i, 0))
    blk = pl.BlockSpec((2 * _BER, _D), lambda i: (i, 0))
    par = pl.BlockSpec((1, _D), lambda i: (0, 0))
    return pl.pallas_call(
        _msg_kernel,
        grid=(grid,),
        in_specs=[blkp, blkp, par, par, par],
        out_specs=blk,
        out_shape=jax.ShapeDtypeStruct((2 * eh, _D), jnp.float32),
    )(ga2, gb2, bm_p.reshape(1, -1), gm_p.reshape(1, -1),
      bem_p.reshape(1, -1))


def _stats_kernel(a_ref, b_ref, o_ref):
    a0, a1 = _unpack2(a_ref[...])
    b0, b1 = _unpack2(b_ref[...])
    q0 = a0 - b0
    q1 = a1 - b1
    blk = jnp.concatenate(
        [jnp.sum(q0, axis=0, keepdims=True)
         + jnp.sum(q1, axis=0, keepdims=True),
         jnp.sum(q0 * q0, axis=0, keepdims=True)
         + jnp.sum(q1 * q1, axis=0, keepdims=True)], axis=0)

    @pl.when(pl.program_id(0) == 0)
    def _init():
        o_ref[...] = jnp.zeros_like(o_ref)

    o_ref[...] += blk


def _stats(ga2, gb2):
    eh = ga2.shape[0]
    blkp = pl.BlockSpec((_BER, _D), lambda i: (i, 0))
    return pl.pallas_call(
        _stats_kernel,
        grid=(eh // _BER,),
        in_specs=[blkp, blkp],
        out_specs=pl.BlockSpec((2, _D), lambda i: (0, 0)),
        out_shape=jax.ShapeDtypeStruct((2, _D), jnp.float32),
    )(ga2, gb2)


def _final_kernel(n_real, a_ref, b_ref, st_ref, g_ref, be_ref, w2_ref, b2_ref,
                  o_ref):
    a0, a1 = _unpack2(a_ref[...])
    b0, b1 = _unpack2(b_ref[...])
    st = st_ref[...]
    m = st[0:1] * (1.0 / n_real)
    v = st[1:2] * (1.0 / n_real) - m * m
    rs = lax.rsqrt(v + _EPS)
    g = g_ref[...]
    be = be_ref[...]
    t0 = _gelu(g * ((a0 - b0) - m) * rs + be)
    t1 = _gelu(g * ((a1 - b1) - m) * rs + be)
    o0 = _mm(t0, w2_ref[...]) + b2_ref[...]
    o1 = _mm(t1, w2_ref[...]) + b2_ref[...]
    o_ref[...] = _ilv(o0, o1)


def _final(ga2, gb2, st, p, perm, n_real):
    eh = ga2.shape[0]
    blkp = pl.BlockSpec((_BER, _D), lambda i: (i, 0))
    par = pl.BlockSpec((1, _D), lambda i: (0, 0))
    w2p = jnp.zeros((8, _D), jnp.float32).at[:3].set(p['fin_W2'][:, perm])
    b2p = jnp.zeros((1, 8), jnp.float32).at[0, :3].set(p['fin_b2'])
    return pl.pallas_call(
        functools.partial(_final_kernel, float(n_real)),
        grid=(eh // _BER,),
        in_specs=[blkp, blkp,
                  pl.BlockSpec((2, _D), lambda i: (0, 0)), par, par,
                  pl.BlockSpec((8, _D), lambda i: (0, 0)),
                  pl.BlockSpec((1, 8), lambda i: (0, 0))],
        out_specs=pl.BlockSpec((2 * _BER, 8), lambda i: (i, 0)),
        out_shape=jax.ShapeDtypeStruct((2 * eh, 8), jnp.float32),
    )(ga2, gb2, st, p['fin_g1'][perm].reshape(1, -1),
      p['fin_be1'][perm].reshape(1, -1), w2p, b2p)


def _sc_mesh():
    return plsc.VectorSubcoreMesh(core_axis_name="c", subcore_axis_name="s",
                                  num_cores=_NC, num_subcores=_NS)


_Q0 = 80
_Q1 = 80


def _sc_gather2(ta, tb, ia, ib):
    e = ia.shape[0]
    assert e == _NS * (_Q0 + _Q1) * _CHUNK
    qmax = max(_Q0, _Q1)
    sh = jax.ShapeDtypeStruct((e, _D // 2), jnp.uint32)

    @functools.partial(
        pl.kernel,
        out_type=(sh, sh),
        mesh=_sc_mesh(),
        compiler_params=pltpu.CompilerParams(use_tc_tiling_on_sc=False),
        scratch_types=[
            pltpu.VMEM((qmax * _CHUNK,), jnp.int32),
            pltpu.VMEM((qmax * _CHUNK,), jnp.int32),
            pltpu.VMEM((_CHUNK, _D // 2), jnp.uint32),
            pltpu.VMEM((_CHUNK, _D // 2), jnp.uint32),
            pltpu.VMEM((_CHUNK, _D // 2), jnp.uint32),
            pltpu.VMEM((_CHUNK, _D // 2), jnp.uint32),
            pltpu.SemaphoreType.DMA,
            pltpu.SemaphoreType.DMA,
            pltpu.SemaphoreType.DMA,
            pltpu.SemaphoreType.DMA,
        ],
    )
    def k(ta_h, tb_h, ia_h, ib_h, oa_h, ob_h,
          iav, ibv, ra0, ra1, rb0, rb1, sg0, sg1, sw0, sw1):
        ra = (ra0, ra1)
        rb = (rb0, rb1)
        sg = (sg0, sg1)
        sw = (sw0, sw1)
        c = lax.axis_index("c")
        s = lax.axis_index("s")

        def drain2(sem):
            pltpu.make_async_copy(ta_h.at[pl.ds(0, _CHUNK)], ra0, sem).wait()
            pltpu.make_async_copy(ta_h.at[pl.ds(0, _CHUNK)], rb0, sem).wait()

        def run(base_e, n_my):
            pltpu.sync_copy(ia_h.at[pl.ds(base_e, n_my * _CHUNK)],
                            iav.at[pl.ds(0, n_my * _CHUNK)])
            pltpu.sync_copy(ib_h.at[pl.ds(base_e, n_my * _CHUNK)],
                            ibv.at[pl.ds(0, n_my * _CHUNK)])

            def start_g(i, b):
                pltpu.async_copy(ta_h.at[iav.at[pl.ds(i * _CHUNK, _CHUNK)]],
                                 ra[b], sg[b])
                pltpu.async_copy(tb_h.at[ibv.at[pl.ds(i * _CHUNK, _CHUNK)]],
                                 rb[b], sg[b])

            def start_wb(i, b):
                base = base_e + i * _CHUNK
                pltpu.async_copy(ra[b], oa_h.at[pl.ds(base, _CHUNK)], sw[b])
                pltpu.async_copy(rb[b], ob_h.at[pl.ds(base, _CHUNK)], sw[b])

            n_pair = n_my // 2
            start_g(0, 0)

            def pair(p, carry):
                @pl.when(p > 0)
                def _():
                    drain2(sw[1])
                start_g(2 * p + 1, 1)
                drain2(sg[0])
                start_wb(2 * p, 0)
                drain2(sw[0])
                @pl.when(p + 1 < n_pair)
                def _():
                    start_g(2 * p + 2, 0)
                drain2(sg[1])
                start_wb(2 * p + 1, 1)
                return carry

            lax.fori_loop(0, n_pair, pair, 0)
            drain2(sw[1])

        @pl.when(c == 0)
        def _():
            run(s * _Q0 * _CHUNK, _Q0)

        @pl.when(c == 1)
        def _():
            run((_NS * _Q0 + s * _Q1) * _CHUNK, _Q1)

    return k(ta, tb, ia, ib)


def _sc_scatter_add(msg, dsts, zrows):
    e = msg.shape[0]
    per_w = e // _NW
    n_ch = per_w // _CHUNK
    zc = _N_ACC // _NS

    @functools.partial(
        pl.kernel,
        out_type=jax.ShapeDtypeStruct((_NC, _N_ACC, _D), jnp.float32),
        mesh=_sc_mesh(),
        scratch_types=[
            pltpu.VMEM((_CHUNK,), jnp.int32),
            pltpu.VMEM((_CHUNK,), jnp.int32),
            pltpu.VMEM((_CHUNK, _D), jnp.float32),
            pltpu.VMEM((_CHUNK, _D), jnp.float32),
            pltpu.SemaphoreType.DMA,
            pltpu.SemaphoreType.DMA,
            pltpu.SemaphoreType.DMA,
            pltpu.SemaphoreType.DMA,
            pltpu.VMEM_SHARED((_N_ACC, _D), jnp.float32),
        ],
    )
    def k(msg_h, dst_h, z_h, out_h, idx0, idx1, rows0, rows1,
          sl0, sl1, ss0, ss1, shared):
        idx = (idx0, idx1)
        rows = (rows0, rows1)
        sl = (sl0, sl1)
        ss = (ss0, ss1)
        c = lax.axis_index("c")
        s = lax.axis_index("s")
        wid = s * _NC + c
        pltpu.sync_copy(z_h.at[pl.ds(s * zc, zc)], shared.at[pl.ds(s * zc, zc)])
        plsc.subcore_barrier()
        base_w = wid * per_w
        n_pair = n_ch // 2

        def start_load(i, b):
            base = base_w + i * _CHUNK
            pltpu.async_copy(dst_h.at[pl.ds(base, _CHUNK)], idx[b], sl[b])
            pltpu.async_copy(msg_h.at[pl.ds(base, _CHUNK)], rows[b], sl[b])

        def drain_load(b):
            pltpu.make_async_copy(dst_h.at[pl.ds(0, _CHUNK)], idx0, sl[b]).wait()
            pltpu.make_async_copy(msg_h.at[pl.ds(0, _CHUNK)], rows0, sl[b]).wait()

        def start_scat(b):
            pltpu.async_copy(rows[b], shared.at[idx[b]], ss[b], add=True)

        def drain_scat(b):
            pltpu.make_async_copy(msg_h.at[pl.ds(0, _CHUNK)], rows0, ss[b]).wait()

        start_load(0, 0)

        def pair(p, carry):
            @pl.when(p > 0)
            def _():
                drain_scat(1)
            start_load(2 * p + 1, 1)
            drain_load(0)
            start_scat(0)
            drain_scat(0)
            @pl.when(p + 1 < n_pair)
            def _():
                start_load(2 * p + 2, 0)
            drain_load(1)
            start_scat(1)
            return carry

        lax.fori_loop(0, n_pair, pair, 0)
        drain_scat(1)
        plsc.subcore_barrier()
        pltpu.sync_copy(shared.at[pl.ds(s * zc, zc)],
                        out_h.at[c, pl.ds(s * zc, zc)])

    return k(msg, dsts, zrows)


def kernel(x, params, edge_index):
    p = params
    n = x.shape[0]
    e = edge_index.shape[1]
    n_ch = (e + _NW * _CHUNK - 1) // (_NW * _CHUNK)
    n_ch += n_ch % 2
    e_pad = _NW * _CHUNK * n_ch
    src = edge_index[0].astype(jnp.int32)
    dst = edge_index[1].astype(jnp.int32)
    pad0 = jnp.zeros((e_pad - e,), jnp.int32)
    ia = jnp.concatenate([dst, pad0])
    ib = jnp.concatenate([src, pad0])
    dsts = jnp.concatenate([dst, jnp.full((e_pad - e,), n, jnp.int32)])
    zrows = jnp.zeros((_N_ACC, _D), jnp.float32)

    perm = jnp.concatenate([jnp.arange(0, _D, 2), jnp.arange(1, _D, 2)])
    h = _node_mlp(x, p)
    h0 = h
    for lp in p['gnn']:
        ai, aj = _proj(h, lp['Wm'][:, :_D][perm, :], lp['Wm'][:, _D:][perm, :])
        ga, gb = _sc_gather2(ai, aj, ia, ib)
        ga2 = ga.reshape(e_pad // 2, _D)
        gb2 = gb.reshape(e_pad // 2, _D)
        msg = _msg(ga2, gb2, lp['bm'][perm], lp['gm'][perm], lp['bem'][perm])
        agg2 = _sc_scatter_add(msg, dsts, zrows)
        h = _update(h, agg2, lp, perm)

    pfin = _mm2(h, h0, p['fin_W1'][perm, :])
    ga, gb = _sc_gather2(pfin, pfin, ia, ib)
    ga2 = ga.reshape(e_pad // 2, _D)
    gb2 = gb.reshape(e_pad // 2, _D)
    st = _stats(ga2, gb2)
    out8 = _final(ga2, gb2, st, p, perm, e)
    return out8[:e, :3]


# speedup vs baseline: 1.3474x; 1.0690x over previous
import functools

import jax
import jax.numpy as jnp
from jax import lax
from jax.experimental import pallas as pl
from jax.experimental.pallas import tpu as pltpu
from jax.experimental.pallas import tpu_sc as plsc

_NC = 2
_NS = 16
_NW = _NC * _NS
_D = 128
_EPS = 1e-5
_CHUNK = 128
_N_ACC = 10112
_BE = 4096
_BER = 2048


def _pack(r):
    lo = lax.bitcast_convert_type(r[:, :64].astype(jnp.bfloat16), jnp.uint16)
    hi = lax.bitcast_convert_type(r[:, 64:].astype(jnp.bfloat16), jnp.uint16)
    return (hi.astype(jnp.uint32) << 16) | lo.astype(jnp.uint32)


def _unpack2(a32):
    lo = lax.bitcast_convert_type(a32 << 16, jnp.float32)
    hi = lax.bitcast_convert_type(a32 & jnp.uint32(0xFFFF0000), jnp.float32)
    e0 = jnp.concatenate([lo[:, :64], hi[:, :64]], axis=-1)
    e1 = jnp.concatenate([lo[:, 64:], hi[:, 64:]], axis=-1)
    return e0, e1


def _ilv(x0, x1):
    m, d = x0.shape
    return jnp.stack([x0, x1], axis=1).reshape(2 * m, d)


def _gelu(t):
    return 0.5 * t * (1.0 + lax.erf(t * 0.7071067811865476))


def _ln_rows(t, g, b):
    m = jnp.mean(t, axis=-1, keepdims=True)
    v = jnp.mean((t - m) ** 2, axis=-1, keepdims=True)
    return g * (t - m) * lax.rsqrt(v + _EPS) + b


def _mm(a, w):
    return lax.dot_general(a, w, (((1,), (1,)), ((), ())),
                           preferred_element_type=jnp.float32)


def _node_mlp_kernel(x_ref, w1_ref, g1_ref, be1_ref, w2_ref, g2_ref, be2_ref,
                     o_ref):
    h = _mm(x_ref[...], w1_ref[...])
    m = jnp.mean(h, axis=0)
    v = jnp.mean((h - m) ** 2, axis=0)
    h = _gelu(g1_ref[...] * (h - m) * lax.rsqrt(v + _EPS) + be1_ref[...])
    h2 = _mm(h, w2_ref[...])
    m2 = jnp.mean(h2, axis=0)
    v2 = jnp.mean((h2 - m2) ** 2, axis=0)
    o_ref[...] = _gelu(g2_ref[...] * (h2 - m2) * lax.rsqrt(v2 + _EPS)
                       + be2_ref[...])


def _node_mlp(x, p):
    n = x.shape[0]
    return pl.pallas_call(
        _node_mlp_kernel,
        out_shape=jax.ShapeDtypeStruct((n, _D), jnp.float32),
    )(x, p['emb_W1'], p['emb_g1'].reshape(1, -1), p['emb_be1'].reshape(1, -1),
      p['emb_W2'], p['emb_g2'].reshape(1, -1), p['emb_be2'].reshape(1, -1))


def _proj_kernel(h_ref, wi_ref, wj_ref, oi_ref, oj_ref):
    oi_ref[...] = _pack(_mm(h_ref[...], wi_ref[...]))
    oj_ref[...] = _pack(_mm(h_ref[...], wj_ref[...]))


def _proj(h, wi, wj):
    n = h.shape[0]
    sh = jax.ShapeDtypeStruct((n, _D // 2), jnp.uint32)
    return pl.pallas_call(_proj_kernel, out_shape=(sh, sh))(h, wi, wj)


def _mm2_kernel(a_ref, b_ref, w_ref, o_ref, o2_ref):
    r = _pack(_mm(a_ref[...] + b_ref[...], w_ref[...]))
    o_ref[...] = r
    o2_ref[...] = r


def _mm2(a, b, w):
    n = a.shape[0]
    sh = jax.ShapeDtypeStruct((n, _D // 2), jnp.uint32)
    return pl.pallas_call(_mm2_kernel, out_shape=(sh, sh))(a, b, w)


def _update_kernel(n, h_ref, ag_ref, wu1_ref, wu2_ref, bu_ref, gu_ref,
                   beu_ref, o_ref):
    h = h_ref[...]
    ag = ag_ref[...]
    agg = ag[0, :n] + ag[1, :n]
    t = _mm(h, wu1_ref[...]) + _mm(agg, wu2_ref[...]) + bu_ref[...]
    o_ref[...] = h + _gelu(_ln_rows(t, gu_ref[...], beu_ref[...]))


def _update(h, agg2, lp, perm):
    n = h.shape[0]
    return pl.pallas_call(
        functools.partial(_update_kernel, n),
        out_shape=jax.ShapeDtypeStruct((n, _D), jnp.float32),
    )(h, agg2, lp['Wu'][:, :_D], lp['Wu'][:, _D:][:, perm],
      lp['bu'].reshape(1, -1), lp['gu'].reshape(1, -1),
      lp['beu'].reshape(1, -1))


def _msg_kernel(a_ref, b_ref, bm_ref, gm_ref, bem_ref, o_ref):
    a0, a1 = _unpack2(a_ref[...])
    b0, b1 = _unpack2(b_ref[...])
    gm = gm_ref[...]
    bem = bem_ref[...]
    m0 = _gelu(_ln_rows(a0 + b0 + bm_ref[...], gm, bem))
    m1 = _gelu(_ln_rows(a1 + b1 + bm_ref[...], gm, bem))
    o_ref[...] = _ilv(m0, m1)


def _msg(ga2, gb2, bm_p, gm_p, bem_p):
    eh = ga2.shape[0]
    grid = eh // _BER
    blkp = pl.BlockSpec((_BER, _D), lambda i: (i, 0))
    blk = pl.BlockSpec((2 * _BER, _D), lambda i: (i, 0))
    par = pl.BlockSpec((1, _D), lambda i: (0, 0))
    return pl.pallas_call(
        _msg_kernel,
        grid=(grid,),
        in_specs=[blkp, blkp, par, par, par],
        out_specs=blk,
        out_shape=jax.ShapeDtypeStruct((2 * eh, _D), jnp.float32),
    )(ga2, gb2, bm_p.reshape(1, -1), gm_p.reshape(1, -1),
      bem_p.reshape(1, -1))


def _stats_kernel(a_ref, b_ref, o_ref):
    a0, a1 = _unpack2(a_ref[...])
    b0, b1 = _unpack2(b_ref[...])
    q0 = a0 - b0
    q1 = a1 - b1
    blk = jnp.concatenate(
        [jnp.sum(q0, axis=0, keepdims=True)
         + jnp.sum(q1, axis=0, keepdims=True),
         jnp.sum(q0 * q0, axis=0, keepdims=True)
         + jnp.sum(q1 * q1, axis=0, keepdims=True)], axis=0)

    @pl.when(pl.program_id(0) == 0)
    def _init():
        o_ref[...] = jnp.zeros_like(o_ref)

    o_ref[...] += blk


def _stats(ga2, gb2):
    eh = ga2.shape[0]
    blkp = pl.BlockSpec((_BER, _D), lambda i: (i, 0))
    return pl.pallas_call(
        _stats_kernel,
        grid=(eh // _BER,),
        in_specs=[blkp, blkp],
        out_specs=pl.BlockSpec((2, _D), lambda i: (0, 0)),
        out_shape=jax.ShapeDtypeStruct((2, _D), jnp.float32),
    )(ga2, gb2)


def _final_kernel(n_real, a_ref, b_ref, st_ref, g_ref, be_ref, w2_ref, b2_ref,
                  o_ref):
    a0, a1 = _unpack2(a_ref[...])
    b0, b1 = _unpack2(b_ref[...])
    st = st_ref[...]
    m = st[0:1] * (1.0 / n_real)
    v = st[1:2] * (1.0 / n_real) - m * m
    rs = lax.rsqrt(v + _EPS)
    g = g_ref[...]
    be = be_ref[...]
    t0 = _gelu(g * ((a0 - b0) - m) * rs + be)
    t1 = _gelu(g * ((a1 - b1) - m) * rs + be)
    o0 = _mm(t0, w2_ref[...]) + b2_ref[...]
    o1 = _mm(t1, w2_ref[...]) + b2_ref[...]
    o_ref[...] = _ilv(o0, o1)


def _final(ga2, gb2, st, p, perm, n_real):
    eh = ga2.shape[0]
    blkp = pl.BlockSpec((_BER, _D), lambda i: (i, 0))
    par = pl.BlockSpec((1, _D), lambda i: (0, 0))
    w2p = jnp.zeros((8, _D), jnp.float32).at[:3].set(p['fin_W2'][:, perm])
    b2p = jnp.zeros((1, 8), jnp.float32).at[0, :3].set(p['fin_b2'])
    return pl.pallas_call(
        functools.partial(_final_kernel, float(n_real)),
        grid=(eh // _BER,),
        in_specs=[blkp, blkp,
                  pl.BlockSpec((2, _D), lambda i: (0, 0)), par, par,
                  pl.BlockSpec((8, _D), lambda i: (0, 0)),
                  pl.BlockSpec((1, 8), lambda i: (0, 0))],
        out_specs=pl.BlockSpec((2 * _BER, 8), lambda i: (i, 0)),
        out_shape=jax.ShapeDtypeStruct((2 * eh, 8), jnp.float32),
    )(ga2, gb2, st, p['fin_g1'][perm].reshape(1, -1),
      p['fin_be1'][perm].reshape(1, -1), w2p, b2p)


def _sc_mesh():
    return plsc.VectorSubcoreMesh(core_axis_name="c", subcore_axis_name="s",
                                  num_cores=_NC, num_subcores=_NS)


_Q0 = 80
_Q1 = 80


def _sc_gather2(ta, tb, ia, ib):
    e = ia.shape[0]
    assert e == _NS * (_Q0 + _Q1) * _CHUNK
    qmax = max(_Q0, _Q1)
    sh = jax.ShapeDtypeStruct((e, _D // 2), jnp.uint32)

    @functools.partial(
        pl.kernel,
        out_type=(sh, sh),
        mesh=_sc_mesh(),
        compiler_params=pltpu.CompilerParams(use_tc_tiling_on_sc=False),
        scratch_types=[
            pltpu.VMEM((qmax * _CHUNK,), jnp.int32),
            pltpu.VMEM((qmax * _CHUNK,), jnp.int32),
            pltpu.VMEM((_CHUNK, _D // 2), jnp.uint32),
            pltpu.VMEM((_CHUNK, _D // 2), jnp.uint32),
            pltpu.VMEM((_CHUNK, _D // 2), jnp.uint32),
            pltpu.VMEM((_CHUNK, _D // 2), jnp.uint32),
            pltpu.SemaphoreType.DMA,
            pltpu.SemaphoreType.DMA,
            pltpu.SemaphoreType.DMA,
            pltpu.SemaphoreType.DMA,
        ],
    )
    def k(ta_h, tb_h, ia_h, ib_h, oa_h, ob_h,
          iav, ibv, ra0, ra1, rb0, rb1, sg0, sg1, sw0, sw1):
        ra = (ra0, ra1)
        rb = (rb0, rb1)
        sg = (sg0, sg1)
        sw = (sw0, sw1)
        c = lax.axis_index("c")
        s = lax.axis_index("s")

        def drain2(sem):
            pltpu.make_async_copy(ta_h.at[pl.ds(0, _CHUNK)], ra0, sem).wait()
            pltpu.make_async_copy(ta_h.at[pl.ds(0, _CHUNK)], rb0, sem).wait()

        def run(base_e, n_my):
            pltpu.sync_copy(ia_h.at[pl.ds(base_e, n_my * _CHUNK)],
                            iav.at[pl.ds(0, n_my * _CHUNK)])
            pltpu.sync_copy(ib_h.at[pl.ds(base_e, n_my * _CHUNK)],
                            ibv.at[pl.ds(0, n_my * _CHUNK)])

            def start_g(i, b):
                pltpu.async_copy(ta_h.at[iav.at[pl.ds(i * _CHUNK, _CHUNK)]],
                                 ra[b], sg[b])
                pltpu.async_copy(tb_h.at[ibv.at[pl.ds(i * _CHUNK, _CHUNK)]],
                                 rb[b], sg[b])

            def start_wb(i, b):
                base = base_e + i * _CHUNK
                pltpu.async_copy(ra[b], oa_h.at[pl.ds(base, _CHUNK)], sw[b])
                pltpu.async_copy(rb[b], ob_h.at[pl.ds(base, _CHUNK)], sw[b])

            n_pair = n_my // 2
            start_g(0, 0)

            def pair(p, carry):
                @pl.when(p > 0)
                def _():
                    drain2(sw[1])
                start_g(2 * p + 1, 1)
                drain2(sg[0])
                start_wb(2 * p, 0)
                drain2(sw[0])
                @pl.when(p + 1 < n_pair)
                def _():
                    start_g(2 * p + 2, 0)
                drain2(sg[1])
                start_wb(2 * p + 1, 1)
                return carry

            lax.fori_loop(0, n_pair, pair, 0)
            drain2(sw[1])

        @pl.when(c == 0)
        def _():
            run(s * _Q0 * _CHUNK, _Q0)

        @pl.when(c == 1)
        def _():
            run((_NS * _Q0 + s * _Q1) * _CHUNK, _Q1)

    return k(ta, tb, ia, ib)


def _sc_scatter_add(msg, dsts, zrows):
    e = msg.shape[0]
    per_w = e // _NW
    n_ch = per_w // _CHUNK
    zc = _N_ACC // _NS

    @functools.partial(
        pl.kernel,
        out_type=jax.ShapeDtypeStruct((_NC, _N_ACC, _D), jnp.float32),
        mesh=_sc_mesh(),
        scratch_types=[
            pltpu.VMEM((_CHUNK,), jnp.int32),
            pltpu.VMEM((_CHUNK,), jnp.int32),
            pltpu.VMEM((_CHUNK, _D), jnp.float32),
            pltpu.VMEM((_CHUNK, _D), jnp.float32),
            pltpu.SemaphoreType.DMA,
            pltpu.SemaphoreType.DMA,
            pltpu.SemaphoreType.DMA,
            pltpu.SemaphoreType.DMA,
            pltpu.VMEM_SHARED((_N_ACC, _D), jnp.float32),
        ],
    )
    def k(msg_h, dst_h, z_h, out_h, idx0, idx1, rows0, rows1,
          sl0, sl1, ss0, ss1, shared):
        idx = (idx0, idx1)
        rows = (rows0, rows1)
        sl = (sl0, sl1)
        ss = (ss0, ss1)
        c = lax.axis_index("c")
        s = lax.axis_index("s")
        wid = s * _NC + c
        pltpu.sync_copy(z_h.at[pl.ds(s * zc, zc)], shared.at[pl.ds(s * zc, zc)])
        plsc.subcore_barrier()
        base_w = wid * per_w
        n_pair = n_ch // 2

        def start_load(i, b):
            base = base_w + i * _CHUNK
            pltpu.async_copy(dst_h.at[pl.ds(base, _CHUNK)], idx[b], sl[b])
            pltpu.async_copy(msg_h.at[pl.ds(base, _CHUNK)], rows[b], sl[b])

        def drain_load(b):
            pltpu.make_async_copy(dst_h.at[pl.ds(0, _CHUNK)], idx0, sl[b]).wait()
            pltpu.make_async_copy(msg_h.at[pl.ds(0, _CHUNK)], rows0, sl[b]).wait()

        def start_scat(b):
            pltpu.async_copy(rows[b], shared.at[idx[b]], ss[b], add=True)

        def drain_scat(b):
            pltpu.make_async_copy(msg_h.at[pl.ds(0, _CHUNK)], rows0, ss[b]).wait()

        start_load(0, 0)

        def pair(p, carry):
            @pl.when(p > 0)
            def _():
                drain_scat(1)
            start_load(2 * p + 1, 1)
            drain_load(0)
            start_scat(0)
            drain_scat(0)
            @pl.when(p + 1 < n_pair)
            def _():
                start_load(2 * p + 2, 0)
            drain_load(1)
            start_scat(1)
            return carry

        lax.fori_loop(0, n_pair, pair, 0)
        drain_scat(1)
        plsc.subcore_barrier()
        pltpu.sync_copy(shared.at[pl.ds(s * zc, zc)],
                        out_h.at[c, pl.ds(s * zc, zc)])

    return k(msg, dsts, zrows)


def kernel(x, params, edge_index):
    p = params
    n = x.shape[0]
    e = edge_index.shape[1]
    n_ch = (e + _NW * _CHUNK - 1) // (_NW * _CHUNK)
    n_ch += n_ch % 2
    e_pad = _NW * _CHUNK * n_ch
    src = edge_index[0].astype(jnp.int32)
    dst = edge_index[1].astype(jnp.int32)
    pad0 = jnp.zeros((e_pad - e,), jnp.int32)
    ia = jnp.concatenate([dst, pad0])
    ib = jnp.concatenate([src, pad0])
    dsts = jnp.concatenate([dst, jnp.full((e_pad - e,), n, jnp.int32)])
    zrows = jnp.zeros((_N_ACC, _D), jnp.float32)

    perm = jnp.concatenate([jnp.arange(0, _D, 2), jnp.arange(1, _D, 2)])
    h = _node_mlp(x, p)
    h0 = h
    for lp in p['gnn']:
        ai, aj = _proj(h, lp['Wm'][:, :_D][perm, :], lp['Wm'][:, _D:][perm, :])
        ga, gb = _sc_gather2(ai, aj, ia, ib)
        ga2 = ga.reshape(e_pad // 2, _D)
        gb2 = gb.reshape(e_pad // 2, _D)
        msg = _msg(ga2, gb2, lp['bm'][perm], lp['gm'][perm], lp['bem'][perm])
        agg2 = _sc_scatter_add(msg, dsts, zrows)
        h = _update(h, agg2, lp, perm)

    pfa, pfb = _mm2(h, h0, p['fin_W1'][perm, :])
    ga, gb = _sc_gather2(pfa, pfb, ia, ib)
    ga2 = ga.reshape(e_pad // 2, _D)
    gb2 = gb.reshape(e_pad // 2, _D)
    st = _stats(ga2, gb2)
    out8 = _final(ga2, gb2, st, p, perm, e)
    return out8[:e, :3]
